# baseline (speedup 1.0000x reference)
import functools

import jax
import jax.numpy as jnp
import numpy as np
from jax import lax
from jax.experimental import pallas as pl
from jax.experimental.pallas import tpu as pltpu
from jax.experimental.pallas import tpu_sc as plsc

NUM_LABELS = 256
B = 8
N = 262144
NC = 2
NS = 16
ROWS_PER_CORE = B // NC
CHUNKS_PER_ROW = NS // ROWS_PER_CORE
CHUNK = N // CHUNKS_PER_ROW
LANES = 16
HSTRIDE = 257
NSUB = 2
SUB = CHUNK // NSUB


def _size_model_body(masks_hbm, out_hbm, data_v, hist_v, merged_v, tmp_v,
                     cnts_v, out_v, shared_sm, sem0, sem1):
    cid = lax.axis_index("c")
    sid = lax.axis_index("s")
    row = cid * ROWS_PER_CORE + sid // CHUNKS_PER_ROW
    q = sid % CHUNKS_PER_ROW

    def start(k, buf):
        return pltpu.async_copy(
            masks_hbm.at[row, pl.ds(q * CHUNK + k * SUB, SUB)],
            data_v.at[pl.ds(buf * SUB, SUB)],
            sem0 if buf == 0 else sem1,
        )

    copies = [start(0, 0), start(1, 1)]

    zeros = jnp.zeros((LANES,), jnp.int32)

    @plsc.parallel_loop(0, HSTRIDE, unroll=8)
    def _(j):
        hist_v[pl.ds(j * LANES, LANES)] = zeros

    lane_base = lax.iota(jnp.int32, LANES) * HSTRIDE
    ones = jnp.ones((LANES,), jnp.int32)

    for k in range(NSUB):
        copies[k].wait()
        base = (k % 2) * SUB

        @plsc.parallel_loop(0, SUB // LANES, unroll=16)
        def _(i):
            v = data_v[pl.ds(base + i * LANES, LANES)]
            plsc.addupdate_scatter(hist_v, [lane_base + v], ones)

        if k + 2 < NSUB:
            copies.append(start(k + 2, k % 2))

    @plsc.parallel_loop(0, NUM_LABELS // LANES, unroll=4)
    def _(c):
        acc = hist_v[pl.ds(c * LANES, LANES)]
        for l in range(1, LANES):
            acc = acc + hist_v[pl.ds(l * HSTRIDE + c * LANES, LANES)]
        merged_v[pl.ds(c * LANES, LANES)] = acc

    pltpu.sync_copy(merged_v, shared_sm.at[pl.ds(sid * NUM_LABELS, NUM_LABELS)])
    plsc.subcore_barrier()

    @pl.when(q == 0)
    def _():
        pltpu.sync_copy(
            shared_sm.at[pl.ds(sid * NUM_LABELS, CHUNKS_PER_ROW * NUM_LABELS)],
            tmp_v)
        lane_iota = lax.iota(jnp.int32, LANES)
        big = jnp.full((LANES,), 1 << 20, jnp.int32)

        @plsc.parallel_loop(0, NUM_LABELS // LANES, unroll=4)
        def _(c):
            acc = tmp_v[pl.ds(c * LANES, LANES)]
            for k in range(1, CHUNKS_PER_ROW):
                acc = acc + tmp_v[pl.ds(k * NUM_LABELS + c * LANES, LANES)]
            acc = jnp.where(c * LANES + lane_iota == 0, big, acc)
            cnts_v[pl.ds(c * LANES, LANES)] = acc

        lo0 = jnp.zeros((LANES,), jnp.int32)
        hi0 = jnp.full((LANES,), N // 128, jnp.int32)

        def step(_, lohi):
            lo, hi = lohi
            mid = (lo + hi) >> 1

            def inner(c, acc):
                ch = cnts_v[pl.ds(c * LANES, LANES)]
                return acc + plsc.all_reduce_population_count(ch <= mid)

            acc = lax.fori_loop(0, NUM_LABELS // LANES, inner,
                                jnp.zeros((LANES,), jnp.int32))
            ge = acc >= 128
            return jnp.where(ge, lo, mid + 1), jnp.where(ge, mid, hi)

        lo, _ = lax.fori_loop(0, 12, step, (lo0, hi0))

        x = lo.astype(jnp.float32)
        seed = jnp.full((LANES,), 0x5F3759DF, jnp.int32)
        y = plsc.bitcast(seed - (plsc.bitcast(x, jnp.int32) >> 1), jnp.float32)
        for _ in range(3):
            y = y * (1.5 - 0.5 * x * y * y)
        s = jnp.where(lo == 0, 0.0, x * y) * np.float32(2.0 / np.sqrt(np.pi))
        out_v[...] = s
        pltpu.sync_copy(out_v, out_hbm.at[row])


_size_model_sc = functools.partial(
    pl.kernel,
    out_type=jax.ShapeDtypeStruct((B, LANES), jnp.float32),
    mesh=plsc.VectorSubcoreMesh(
        core_axis_name="c", subcore_axis_name="s", num_cores=NC, num_subcores=NS
    ),
    scratch_types=[
        pltpu.VMEM((CHUNK,), jnp.int32),
        pltpu.VMEM((LANES * HSTRIDE,), jnp.int32),
        pltpu.VMEM((NUM_LABELS,), jnp.int32),
        pltpu.VMEM((CHUNKS_PER_ROW * NUM_LABELS,), jnp.int32),
        pltpu.VMEM((NUM_LABELS,), jnp.int32),
        pltpu.VMEM((LANES,), jnp.float32),
        pltpu.VMEM_SHARED((NS * NUM_LABELS,), jnp.int32),
        pltpu.SemaphoreType.DMA,
        pltpu.SemaphoreType.DMA,
    ],
    compiler_params=pltpu.CompilerParams(needs_layout_passes=False),
)(_size_model_body)


def kernel(masks):
    out = _size_model_sc(masks)
    return out[:, 0]


# speedup vs baseline: 1.0354x; 1.0354x over previous
import functools

import jax
import jax.numpy as jnp
import numpy as np
from jax import lax
from jax.experimental import pallas as pl
from jax.experimental.pallas import tpu as pltpu
from jax.experimental.pallas import tpu_sc as plsc

NUM_LABELS = 256
B = 8
N = 262144
NC = 2
NS = 16
ROWS_PER_CORE = B // NC
CHUNKS_PER_ROW = NS // ROWS_PER_CORE
CHUNK = N // CHUNKS_PER_ROW
LANES = 16
HSTRIDE = 257
NSUB = 4
SUB = CHUNK // NSUB


def _size_model_body(masks_hbm, out_hbm, data_v, hist_v, merged_v, tmp_v,
                     cnts_v, out_v, shared_sm, sem0, sem1):
    cid = lax.axis_index("c")
    sid = lax.axis_index("s")
    row = cid * ROWS_PER_CORE + sid // CHUNKS_PER_ROW
    q = sid % CHUNKS_PER_ROW

    def start(k, buf):
        return pltpu.async_copy(
            masks_hbm.at[row, pl.ds(q * CHUNK + k * SUB, SUB)],
            data_v.at[pl.ds(buf * SUB, SUB)],
            sem0 if buf == 0 else sem1,
        )

    copies = [start(0, 0), start(1, 1)]

    zeros = jnp.zeros((LANES,), jnp.int32)

    @plsc.parallel_loop(0, HSTRIDE, unroll=8)
    def _(j):
        hist_v[pl.ds(j * LANES, LANES)] = zeros

    lane_base = lax.iota(jnp.int32, LANES) * HSTRIDE
    ones = jnp.ones((LANES,), jnp.int32)

    for k in range(NSUB):
        copies[k].wait()
        base = (k % 2) * SUB

        @plsc.parallel_loop(0, SUB // LANES, unroll=16)
        def _(i):
            v = data_v[pl.ds(base + i * LANES, LANES)]
            plsc.addupdate_scatter(hist_v, [lane_base + v], ones)

        if k + 2 < NSUB:
            copies.append(start(k + 2, k % 2))

    @plsc.parallel_loop(0, NUM_LABELS // LANES, unroll=4)
    def _(c):
        acc = hist_v[pl.ds(c * LANES, LANES)]
        for l in range(1, LANES):
            acc = acc + hist_v[pl.ds(l * HSTRIDE + c * LANES, LANES)]
        merged_v[pl.ds(c * LANES, LANES)] = acc

    pltpu.sync_copy(merged_v, shared_sm.at[pl.ds(sid * NUM_LABELS, NUM_LABELS)])
    plsc.subcore_barrier()

    @pl.when(q == 0)
    def _():
        pltpu.sync_copy(
            shared_sm.at[pl.ds(sid * NUM_LABELS, CHUNKS_PER_ROW * NUM_LABELS)],
            tmp_v)
        lane_iota = lax.iota(jnp.int32, LANES)
        big = jnp.full((LANES,), 1 << 20, jnp.int32)

        @plsc.parallel_loop(0, NUM_LABELS // LANES, unroll=4)
        def _(c):
            acc = tmp_v[pl.ds(c * LANES, LANES)]
            for k in range(1, CHUNKS_PER_ROW):
                acc = acc + tmp_v[pl.ds(k * NUM_LABELS + c * LANES, LANES)]
            acc = jnp.where(c * LANES + lane_iota == 0, big, acc)
            cnts_v[pl.ds(c * LANES, LANES)] = acc

        lo0 = jnp.zeros((LANES,), jnp.int32)
        hi0 = jnp.full((LANES,), N // 128, jnp.int32)

        def step(_, lohi):
            lo, hi = lohi
            mid = (lo + hi) >> 1

            def inner(c, acc):
                ch = cnts_v[pl.ds(c * LANES, LANES)]
                return acc + plsc.all_reduce_population_count(ch <= mid)

            acc = lax.fori_loop(0, NUM_LABELS // LANES, inner,
                                jnp.zeros((LANES,), jnp.int32))
            ge = acc >= 128
            return jnp.where(ge, lo, mid + 1), jnp.where(ge, mid, hi)

        lo, _ = lax.fori_loop(0, 12, step, (lo0, hi0))

        x = lo.astype(jnp.float32)
        seed = jnp.full((LANES,), 0x5F3759DF, jnp.int32)
        y = plsc.bitcast(seed - (plsc.bitcast(x, jnp.int32) >> 1), jnp.float32)
        for _ in range(3):
            y = y * (1.5 - 0.5 * x * y * y)
        s = jnp.where(lo == 0, 0.0, x * y) * np.float32(2.0 / np.sqrt(np.pi))
        out_v[...] = s
        pltpu.sync_copy(out_v, out_hbm.at[row])


_size_model_sc = functools.partial(
    pl.kernel,
    out_type=jax.ShapeDtypeStruct((B, LANES), jnp.float32),
    mesh=plsc.VectorSubcoreMesh(
        core_axis_name="c", subcore_axis_name="s", num_cores=NC, num_subcores=NS
    ),
    scratch_types=[
        pltpu.VMEM((CHUNK,), jnp.int32),
        pltpu.VMEM((LANES * HSTRIDE,), jnp.int32),
        pltpu.VMEM((NUM_LABELS,), jnp.int32),
        pltpu.VMEM((CHUNKS_PER_ROW * NUM_LABELS,), jnp.int32),
        pltpu.VMEM((NUM_LABELS,), jnp.int32),
        pltpu.VMEM((LANES,), jnp.float32),
        pltpu.VMEM_SHARED((NS * NUM_LABELS,), jnp.int32),
        pltpu.SemaphoreType.DMA,
        pltpu.SemaphoreType.DMA,
    ],
    compiler_params=pltpu.CompilerParams(needs_layout_passes=False),
)(_size_model_body)


def kernel(masks):
    out = _size_model_sc(masks)
    return out[:, 0]
